# baseline (speedup 1.0000x reference)
import functools

import jax
import jax.numpy as jnp
from jax import lax
from jax.experimental import pallas as pl
from jax.experimental.pallas import tpu as pltpu
from jax.experimental.pallas import tpu_sc as plsc

_H = 4096
_W = 4096
_HW = _H * _W
_P = 500000
_THETA = 0.5

_NC = 2
_NS = 16
_NW = _NC * _NS

_PPAD = 503808
_G = _PPAD // _NW
_GH = _G // 2
_NI = 2 * _G
_CHUNK = 128
_NCH = _NI // _CHUNK
_INFLIGHT = 32

_NBUK = 512
_BSH = 15
_NSUB = _NBUK * 16


def _copy_body(x_ref, o_ref):
    o_ref[...] = x_ref[...]


_tc_copy = pl.pallas_call(
    _copy_body,
    out_shape=jax.ShapeDtypeStruct((_H, _W), jnp.float32),
    grid=(32,),
    in_specs=[pl.BlockSpec((_H // 32, _W), lambda i: (i, 0))],
    out_specs=pl.BlockSpec((_H // 32, _W), lambda i: (i, 0)),
)


def _sc_body(out_hbm, r0_h, c0_h, r1_h, c1_h, b_h, d_h,
             r0_v, c0_v, r1_v, c1_v, b_v, d_v,
             raw_v, idx2d, hist, zeros_v, sem):
    wid = lax.axis_index("s") * _NC + lax.axis_index("c")
    base = wid * _G

    iota = lax.iota(jnp.int32, 16)
    ones16 = jnp.ones((16,), jnp.int32)
    zero16 = jnp.zeros((16,), jnp.int32)

    def hz_body(i, _):
        for j in range(8):
            hist[pl.ds((i * 8 + j) * 16, 16)] = zero16
        return jnp.int32(0)

    lax.fori_loop(0, _NSUB // 128, hz_body, jnp.int32(0))

    for j in range(_CHUNK // 16):
        zeros_v[pl.ds(j * 16, 16)] = jnp.zeros((16,), jnp.float32)

    stage = [(r0_h, r0_v), (c0_h, c0_v), (r1_h, r1_v),
             (c1_h, c1_v), (b_h, b_v), (d_h, d_v)]
    for ch in range(2):
        off = base + ch * _GH
        for src, dst in stage:
            pltpu.async_copy(src.at[pl.ds(off, _GH)], dst, sem)
        for src, dst in stage:
            pltpu.make_async_copy(src.at[pl.ds(off, _GH)], dst, sem).wait()

        def p1_body(i, _):
            for j in range(4):
                v = i * 4 + j
                sl = pl.ds(v * 16, 16)
                m = jnp.abs(d_v[sl] - b_v[sl]) <= _THETA
                f0 = jnp.where(m, r0_v[sl] * _W + c0_v[sl], -1)
                f1 = jnp.where(m, r1_v[sl] * _W + c1_v[sl], -1)
                raw_v[pl.ds(ch * 2 * _GH + v * 16, 16)] = f0
                raw_v[pl.ds(ch * 2 * _GH + _GH + v * 16, 16)] = f1
                s0 = lax.shift_right_arithmetic(f0, _BSH) * 16 + iota
                s1 = lax.shift_right_arithmetic(f1, _BSH) * 16 + iota
                plsc.addupdate_scatter(hist, [jnp.where(m, s0, 0)], ones16,
                                       mask=m)
                plsc.addupdate_scatter(hist, [jnp.where(m, s1, 0)], ones16,
                                       mask=m)
            return jnp.int32(0)

        lax.fori_loop(0, _GH // 64, p1_body, jnp.int32(0))

    def off_body(i, carry):
        for j in range(4):
            sl = pl.ds((i * 4 + j) * 16, 16)
            h = hist[sl]
            inc = plsc.cumsum(h)
            hist[sl] = inc - h + carry
            carry = carry + jnp.sum(h)
        return carry

    nvalid = lax.fori_loop(0, _NSUB // 64, off_body, jnp.int32(0))

    @pl.when(nvalid < _NI)
    def _():
        row = lax.shift_right_logical(nvalid, 7)
        for j in range(8):
            idx2d[row, pl.ds(j * 16, 16)] = zero16 - 1

    def p2_body(i, _):
        for j in range(4):
            sl = pl.ds((i * 4 + j) * 16, 16)
            f = raw_v[sl]
            m = f >= 0
            sub = jnp.where(m, lax.shift_right_arithmetic(f, _BSH) * 16 + iota,
                            0)
            pos = plsc.load_gather(hist, [sub], mask=m)
            plsc.addupdate_scatter(hist, [sub], ones16, mask=m)
            row = lax.shift_right_logical(pos, 7)
            col = lax.bitwise_and(pos, 127)
            plsc.store_scatter(idx2d, [row, col], f, mask=m)
        return jnp.int32(0)

    lax.fori_loop(0, _NI // 64, p2_body, jnp.int32(0))

    nrows = lax.shift_right_logical(nvalid + 127, 7)

    def _chunk_copy(c):
        return pltpu.make_async_copy(
            zeros_v, out_hbm.at[plsc.Indices(idx2d.at[c], ignored_value=-1)],
            sem)

    def fire_body(c, _):
        _chunk_copy(c).start()

        @pl.when(c >= _INFLIGHT)
        def _():
            _chunk_copy(c - _INFLIGHT).wait()

        return jnp.int32(0)

    lax.fori_loop(0, nrows, fire_body, jnp.int32(0))

    def drain_body(c, _):
        _chunk_copy(c).wait()
        return jnp.int32(0)

    lax.fori_loop(jnp.maximum(nrows - _INFLIGHT, 0), nrows, drain_body,
                  jnp.int32(0))


@functools.cache
def _sc_scatter():
    mesh = plsc.VectorSubcoreMesh(core_axis_name="c", subcore_axis_name="s")
    return pl.kernel(
        _sc_body,
        out_type=(),
        mesh=mesh,
        compiler_params=pltpu.CompilerParams(needs_layout_passes=False),
        scratch_types=[
            pltpu.VMEM((_GH,), jnp.int32),
            pltpu.VMEM((_GH,), jnp.int32),
            pltpu.VMEM((_GH,), jnp.int32),
            pltpu.VMEM((_GH,), jnp.int32),
            pltpu.VMEM((_GH,), jnp.float32),
            pltpu.VMEM((_GH,), jnp.float32),
            pltpu.VMEM((_NI,), jnp.int32),
            pltpu.VMEM((_NCH, _CHUNK), jnp.int32),
            pltpu.VMEM((_NSUB,), jnp.int32),
            pltpu.VMEM((_CHUNK,), jnp.float32),
            pltpu.SemaphoreType.DMA,
        ],
    )


def kernel(x, gens, pd):
    npad = _PPAD - _P
    zpad = jnp.zeros((npad,), jnp.int32)
    cols = [jnp.concatenate([gens[:, j], zpad]) for j in range(4)]
    pd_b = jnp.concatenate([pd[:, 0], jnp.zeros((npad,), jnp.float32)])
    pd_d = jnp.concatenate([pd[:, 1], jnp.ones((npad,), jnp.float32)])

    out = _tc_copy(x).reshape(_HW)
    out_ref = jax.new_ref(out)
    _sc_scatter()(out_ref, *cols, pd_b, pd_d)
    return out_ref[...].reshape(_H, _W)


# speedup vs baseline: 3.1233x; 3.1233x over previous
import functools

import jax
import jax.numpy as jnp
from jax import lax
from jax.experimental import pallas as pl
from jax.experimental.pallas import tpu as pltpu
from jax.experimental.pallas import tpu_sc as plsc

_H = 4096
_W = 4096
_HW = _H * _W
_P = 500000
_THETA = 0.5

_NC = 2
_NS = 16
_NW = _NC * _NS

_PPAD = 503808
_G = _PPAD // _NW
_GH = _G // 2
_NI = 2 * _G

_NBUK = 512
_BSH = 15
_BCELLS = 1 << _BSH
_KPW = _NBUK // _NW
_NSUB = _NBUK * 16

_SSTR = _NI + 128
_BSTR = 528
_CH = 128


def _sc_sort_body(r0_h, c0_h, r1_h, c1_h, b_h, d_h, sorted_h, bounds_h,
                  r0_v, c0_v, r1_v, c1_v, b_v, d_v,
                  raw_v, srt_v, hist, bnd_v, sem):
    wid = lax.axis_index("s") * _NC + lax.axis_index("c")
    base = wid * _G

    iota = lax.iota(jnp.int32, 16)
    ones16 = jnp.ones((16,), jnp.int32)
    zero16 = jnp.zeros((16,), jnp.int32)

    def hz_body(i, _):
        for j in range(8):
            hist[pl.ds((i * 8 + j) * 16, 16)] = zero16
        return jnp.int32(0)

    lax.fori_loop(0, _NSUB // 128, hz_body, jnp.int32(0))

    stage = [(r0_h, r0_v), (c0_h, c0_v), (r1_h, r1_v),
             (c1_h, c1_v), (b_h, b_v), (d_h, d_v)]
    for ch in range(2):
        off = base + ch * _GH
        for src, dst in stage:
            pltpu.async_copy(src.at[pl.ds(off, _GH)], dst, sem)
        for src, dst in stage:
            pltpu.make_async_copy(src.at[pl.ds(off, _GH)], dst, sem).wait()

        def p1_body(i, _):
            for j in range(4):
                v = i * 4 + j
                sl = pl.ds(v * 16, 16)
                m = jnp.abs(d_v[sl] - b_v[sl]) <= _THETA
                f0 = jnp.where(m, r0_v[sl] * _W + c0_v[sl], -1)
                f1 = jnp.where(m, r1_v[sl] * _W + c1_v[sl], -1)
                raw_v[pl.ds(ch * 2 * _GH + v * 16, 16)] = f0
                raw_v[pl.ds(ch * 2 * _GH + _GH + v * 16, 16)] = f1
                s0 = lax.shift_right_arithmetic(f0, _BSH) * 16 + iota
                s1 = lax.shift_right_arithmetic(f1, _BSH) * 16 + iota
                plsc.addupdate_scatter(hist, [jnp.where(m, s0, 0)], ones16,
                                       mask=m)
                plsc.addupdate_scatter(hist, [jnp.where(m, s1, 0)], ones16,
                                       mask=m)
            return jnp.int32(0)

        lax.fori_loop(0, _GH // 64, p1_body, jnp.int32(0))

    def off_body(i, carry):
        for j in range(4):
            sl = pl.ds((i * 4 + j) * 16, 16)
            h = hist[sl]
            inc = plsc.cumsum(h)
            hist[sl] = inc - h + carry
            carry = carry + jnp.sum(h)
        return carry

    nvalid = lax.fori_loop(0, _NSUB // 64, off_body, jnp.int32(0))

    def bb_body(i, _):
        kvec = i * 16 + iota
        bnd_v[pl.ds(i * 16, 16)] = plsc.load_gather(hist, [kvec * 16])
        return jnp.int32(0)

    lax.fori_loop(0, _NBUK // 16, bb_body, jnp.int32(0))
    m0 = iota == 0
    plsc.store_scatter(bnd_v, [jnp.where(m0, _NBUK, 0)],
                       jnp.full((16,), 0, jnp.int32) + nvalid, mask=m0)

    def p2_body(i, _):
        for j in range(4):
            sl = pl.ds((i * 4 + j) * 16, 16)
            f = raw_v[sl]
            m = f >= 0
            sub = jnp.where(m, lax.shift_right_arithmetic(f, _BSH) * 16 + iota,
                            0)
            pos = plsc.load_gather(hist, [sub], mask=m)
            plsc.addupdate_scatter(hist, [sub], ones16, mask=m)
            plsc.store_scatter(srt_v, [jnp.where(m, pos, 0)], f, mask=m)
        return jnp.int32(0)

    lax.fori_loop(0, _NI // 64, p2_body, jnp.int32(0))

    pltpu.sync_copy(srt_v, sorted_h.at[pl.ds(wid * _SSTR, _NI)])
    pltpu.sync_copy(bnd_v, bounds_h.at[pl.ds(wid * _BSTR, _BSTR)])


def _sc_apply_body(x_h, sorted_h, bounds_h, out_h,
                   bnd_v, reg_v, chk_v, st_v, en_v, sem, semr):
    wid = lax.axis_index("s") * _NC + lax.axis_index("c")
    kbase = wid * _KPW

    iota = lax.iota(jnp.int32, 16)
    zerosf = jnp.zeros((16,), jnp.float32)

    for s in range(_NW):
        pltpu.async_copy(bounds_h.at[pl.ds(s * _BSTR + kbase, 32)],
                         bnd_v.at[pl.ds(s * 32, 32)], sem)
    for s in range(_NW):
        pltpu.make_async_copy(bounds_h.at[pl.ds(s * _BSTR + kbase, 32)],
                              bnd_v.at[pl.ds(s * 32, 32)], sem).wait()

    def _extract(buf, vbase, p):
        v = buf[pl.ds(vbase + lax.bitwise_and(p, 16), 16)]
        return jnp.sum(jnp.where(iota == lax.bitwise_and(p, 15), v, 0))

    def bucket_body(k, _):
        rbase = pl.multiple_of((kbase + k) * _BCELLS, _BCELLS)
        pltpu.async_copy(x_h.at[pl.ds(rbase, _BCELLS)], reg_v, semr)

        def seg_issue(s, _):
            start = _extract(bnd_v, s * 32, k)
            end = _extract(bnd_v, s * 32, k + 1)
            astart = lax.bitwise_and(start, jnp.int32(-8))
            plsc.store_scatter(st_v, [jnp.where(iota == 0, s, 0)],
                               jnp.zeros((16,), jnp.int32) + start,
                               mask=iota == 0)
            plsc.store_scatter(en_v, [jnp.where(iota == 0, s, 0)],
                               jnp.zeros((16,), jnp.int32) + end,
                               mask=iota == 0)
            soff = pl.multiple_of(s * _SSTR + astart, 8)
            pltpu.async_copy(sorted_h.at[pl.ds(soff, _CH)],
                             chk_v.at[s], sem)
            return jnp.int32(0)

        lax.fori_loop(0, _NW, seg_issue, jnp.int32(0))

        def seg_wait(s, _):
            start = _extract(st_v, 0, s)
            astart = lax.bitwise_and(start, jnp.int32(-8))
            soff = pl.multiple_of(s * _SSTR + astart, 8)
            pltpu.make_async_copy(sorted_h.at[pl.ds(soff, _CH)],
                                  chk_v.at[s], sem).wait()
            return jnp.int32(0)

        lax.fori_loop(0, _NW, seg_wait, jnp.int32(0))
        pltpu.make_async_copy(x_h.at[pl.ds(rbase, _BCELLS)], reg_v,
                              semr).wait()

        def seg_apply(s, _):
            start = _extract(st_v, 0, s)
            end = _extract(en_v, 0, s)
            astart = lax.bitwise_and(start, jnp.int32(-8))
            nch = lax.shift_right_logical(end - astart + (_CH - 1), 7)

            def ch_body(c, _):
                coff = astart + c * _CH

                @pl.when(c > 0)
                def _():
                    soff = pl.multiple_of(s * _SSTR + coff, 8)
                    pltpu.async_copy(
                        sorted_h.at[pl.ds(soff, _CH)],
                        chk_v.at[s], sem).wait()

                for v in range(_CH // 16):
                    posa = coff + v * 16 + iota
                    f = chk_v[s, pl.ds(v * 16, 16)]
                    m = jnp.logical_and(posa >= start, posa < end)
                    loc = jnp.where(m, f - rbase, 0)
                    plsc.store_scatter(reg_v, [loc], zerosf, mask=m)
                return jnp.int32(0)

            lax.fori_loop(0, nch, ch_body, jnp.int32(0))
            return jnp.int32(0)

        lax.fori_loop(0, _NW, seg_apply, jnp.int32(0))

        pltpu.sync_copy(reg_v, out_h.at[pl.ds(rbase, _BCELLS)])
        return jnp.int32(0)

    lax.fori_loop(0, _KPW, bucket_body, jnp.int32(0))


@functools.cache
def _sc_sort():
    mesh = plsc.VectorSubcoreMesh(core_axis_name="c", subcore_axis_name="s")
    return pl.kernel(
        _sc_sort_body,
        out_type=(
            jax.ShapeDtypeStruct((_NW * _SSTR,), jnp.int32),
            jax.ShapeDtypeStruct((_NW * _BSTR,), jnp.int32),
        ),
        mesh=mesh,
        compiler_params=pltpu.CompilerParams(needs_layout_passes=False),
        scratch_types=[
            pltpu.VMEM((_GH,), jnp.int32),
            pltpu.VMEM((_GH,), jnp.int32),
            pltpu.VMEM((_GH,), jnp.int32),
            pltpu.VMEM((_GH,), jnp.int32),
            pltpu.VMEM((_GH,), jnp.float32),
            pltpu.VMEM((_GH,), jnp.float32),
            pltpu.VMEM((_NI,), jnp.int32),
            pltpu.VMEM((_NI,), jnp.int32),
            pltpu.VMEM((_NSUB,), jnp.int32),
            pltpu.VMEM((_BSTR,), jnp.int32),
            pltpu.SemaphoreType.DMA,
        ],
    )


@functools.cache
def _sc_apply():
    mesh = plsc.VectorSubcoreMesh(core_axis_name="c", subcore_axis_name="s")
    return pl.kernel(
        _sc_apply_body,
        out_type=jax.ShapeDtypeStruct((_HW,), jnp.float32),
        mesh=mesh,
        compiler_params=pltpu.CompilerParams(needs_layout_passes=False),
        scratch_types=[
            pltpu.VMEM((_NW * 32,), jnp.int32),
            pltpu.VMEM((_BCELLS,), jnp.float32),
            pltpu.VMEM((_NW, _CH), jnp.int32),
            pltpu.VMEM((_NW,), jnp.int32),
            pltpu.VMEM((_NW,), jnp.int32),
            pltpu.SemaphoreType.DMA,
            pltpu.SemaphoreType.DMA,
        ],
    )


def kernel(x, gens, pd):
    npad = _PPAD - _P
    zpad = jnp.zeros((npad,), jnp.int32)
    cols = [jnp.concatenate([gens[:, j], zpad]) for j in range(4)]
    pd_b = jnp.concatenate([pd[:, 0], jnp.zeros((npad,), jnp.float32)])
    pd_d = jnp.concatenate([pd[:, 1], jnp.ones((npad,), jnp.float32)])

    sorted_h, bounds_h = _sc_sort()(*cols, pd_b, pd_d)
    out = _sc_apply()(x.reshape(_HW), sorted_h, bounds_h)
    return out.reshape(_H, _W)


# speedup vs baseline: 4.0389x; 1.2931x over previous
import functools

import jax
import jax.numpy as jnp
from jax import lax
from jax.experimental import pallas as pl
from jax.experimental.pallas import tpu as pltpu
from jax.experimental.pallas import tpu_sc as plsc

_H = 4096
_W = 4096
_HW = _H * _W
_P = 500000
_THETA = 0.5

_NC = 2
_NS = 16
_NW = _NC * _NS

_PPAD = 503808
_G = _PPAD // _NW
_GH = _G // 2
_NI = 2 * _G

_NBUK = 512
_BSH = 15
_BCELLS = 1 << _BSH
_KPW = _NBUK // _NW
_NSUB = _NBUK * 16

_SSTR = _NI + 128
_BSTR = 528
_CH = 128


def _sc_sort_body(r0_h, c0_h, r1_h, c1_h, b_h, d_h, sorted_h, bounds_h,
                  r0_v, c0_v, r1_v, c1_v, b_v, d_v,
                  raw_v, srt_v, hist, bnd_v, sem):
    wid = lax.axis_index("s") * _NC + lax.axis_index("c")
    base = wid * _G

    iota = lax.iota(jnp.int32, 16)
    ones16 = jnp.ones((16,), jnp.int32)
    zero16 = jnp.zeros((16,), jnp.int32)

    def hz_body(i, _):
        for j in range(8):
            hist[pl.ds((i * 8 + j) * 16, 16)] = zero16
        return jnp.int32(0)

    lax.fori_loop(0, _NSUB // 128, hz_body, jnp.int32(0))

    stage = [(r0_h, r0_v), (c0_h, c0_v), (r1_h, r1_v),
             (c1_h, c1_v), (b_h, b_v), (d_h, d_v)]
    for ch in range(2):
        off = base + ch * _GH
        for src, dst in stage:
            pltpu.async_copy(src.at[pl.ds(off, _GH)], dst, sem)
        for src, dst in stage:
            pltpu.make_async_copy(src.at[pl.ds(off, _GH)], dst, sem).wait()

        def p1_body(i, _):
            for j in range(4):
                v = i * 4 + j
                sl = pl.ds(v * 16, 16)
                m = jnp.abs(d_v[sl] - b_v[sl]) <= _THETA
                f0 = jnp.where(m, r0_v[sl] * _W + c0_v[sl], -1)
                f1 = jnp.where(m, r1_v[sl] * _W + c1_v[sl], -1)
                raw_v[pl.ds(ch * 2 * _GH + v * 16, 16)] = f0
                raw_v[pl.ds(ch * 2 * _GH + _GH + v * 16, 16)] = f1
                s0 = lax.shift_right_arithmetic(f0, _BSH) * 16 + iota
                s1 = lax.shift_right_arithmetic(f1, _BSH) * 16 + iota
                plsc.addupdate_scatter(hist, [jnp.where(m, s0, 0)], ones16,
                                       mask=m)
                plsc.addupdate_scatter(hist, [jnp.where(m, s1, 0)], ones16,
                                       mask=m)
            return jnp.int32(0)

        lax.fori_loop(0, _GH // 64, p1_body, jnp.int32(0))

    def off_body(i, carry):
        for j in range(4):
            sl = pl.ds((i * 4 + j) * 16, 16)
            h = hist[sl]
            inc = plsc.cumsum(h)
            hist[sl] = inc - h + carry
            carry = carry + jnp.sum(h)
        return carry

    nvalid = lax.fori_loop(0, _NSUB // 64, off_body, jnp.int32(0))

    def bb_body(i, _):
        kvec = i * 16 + iota
        bnd_v[pl.ds(i * 16, 16)] = plsc.load_gather(hist, [kvec * 16])
        return jnp.int32(0)

    lax.fori_loop(0, _NBUK // 16, bb_body, jnp.int32(0))
    m0 = iota == 0
    plsc.store_scatter(bnd_v, [jnp.where(m0, _NBUK, 0)],
                       jnp.full((16,), 0, jnp.int32) + nvalid, mask=m0)

    def p2_body(i, _):
        for j in range(4):
            sl = pl.ds((i * 4 + j) * 16, 16)
            f = raw_v[sl]
            m = f >= 0
            sub = jnp.where(m, lax.shift_right_arithmetic(f, _BSH) * 16 + iota,
                            0)
            pos = plsc.load_gather(hist, [sub], mask=m)
            plsc.addupdate_scatter(hist, [sub], ones16, mask=m)
            plsc.store_scatter(srt_v, [jnp.where(m, pos, 0)], f, mask=m)
        return jnp.int32(0)

    lax.fori_loop(0, _NI // 64, p2_body, jnp.int32(0))

    pltpu.sync_copy(srt_v, sorted_h.at[pl.ds(wid * _SSTR, _NI)])
    pltpu.sync_copy(bnd_v, bounds_h.at[pl.ds(wid * _BSTR, _BSTR)])


def _sc_apply_body(x_h, sorted_h, bounds_h, out_h,
                   bnd_v, reg_v, chk_v, st_v, en_v, sem, semr):
    wid = lax.axis_index("s") * _NC + lax.axis_index("c")
    kbase = wid * _KPW

    iota = lax.iota(jnp.int32, 16)
    zerosf = jnp.zeros((16,), jnp.float32)

    for s in range(_NW):
        pltpu.async_copy(bounds_h.at[pl.ds(s * _BSTR + kbase, 32)],
                         bnd_v.at[pl.ds(s * 32, 32)], sem)
    for s in range(_NW):
        pltpu.make_async_copy(bounds_h.at[pl.ds(s * _BSTR + kbase, 32)],
                              bnd_v.at[pl.ds(s * 32, 32)], sem).wait()

    def _extract(buf, vbase, p):
        v = buf[pl.ds(vbase + lax.bitwise_and(p, 16), 16)]
        return jnp.sum(jnp.where(iota == lax.bitwise_and(p, 15), v, 0))

    def bucket_body(k, _):
        rbase = (kbase + k) * _BCELLS
        row0 = pl.multiple_of((kbase + k) * (_BCELLS // _W), 8)
        pltpu.async_copy(x_h.at[pl.ds(row0, _BCELLS // _W)], reg_v, semr)

        def seg_issue(s, _):
            start = _extract(bnd_v, s * 32, k)
            end = _extract(bnd_v, s * 32, k + 1)
            astart = lax.bitwise_and(start, jnp.int32(-8))
            plsc.store_scatter(st_v, [jnp.where(iota == 0, s, 0)],
                               jnp.zeros((16,), jnp.int32) + start,
                               mask=iota == 0)
            plsc.store_scatter(en_v, [jnp.where(iota == 0, s, 0)],
                               jnp.zeros((16,), jnp.int32) + end,
                               mask=iota == 0)
            soff = pl.multiple_of(s * _SSTR + astart, 8)
            pltpu.async_copy(sorted_h.at[pl.ds(soff, _CH)],
                             chk_v.at[s], sem)
            return jnp.int32(0)

        lax.fori_loop(0, _NW, seg_issue, jnp.int32(0))

        def seg_wait(s, _):
            start = _extract(st_v, 0, s)
            astart = lax.bitwise_and(start, jnp.int32(-8))
            soff = pl.multiple_of(s * _SSTR + astart, 8)
            pltpu.make_async_copy(sorted_h.at[pl.ds(soff, _CH)],
                                  chk_v.at[s], sem).wait()
            return jnp.int32(0)

        lax.fori_loop(0, _NW, seg_wait, jnp.int32(0))
        pltpu.make_async_copy(x_h.at[pl.ds(row0, _BCELLS // _W)], reg_v,
                              semr).wait()

        def seg_apply(s, _):
            start = _extract(st_v, 0, s)
            end = _extract(en_v, 0, s)
            astart = lax.bitwise_and(start, jnp.int32(-8))
            nch = lax.shift_right_logical(end - astart + (_CH - 1), 7)

            def ch_body(c, _):
                coff = astart + c * _CH

                @pl.when(c > 0)
                def _():
                    soff = pl.multiple_of(s * _SSTR + coff, 8)
                    pltpu.async_copy(
                        sorted_h.at[pl.ds(soff, _CH)],
                        chk_v.at[s], sem).wait()

                for v in range(_CH // 16):
                    posa = coff + v * 16 + iota
                    f = chk_v[s, pl.ds(v * 16, 16)]
                    m = jnp.logical_and(posa >= start, posa < end)
                    loc = jnp.where(m, f - rbase, 0)
                    lr = lax.shift_right_logical(loc, 12)
                    lc = lax.bitwise_and(loc, _W - 1)
                    plsc.store_scatter(reg_v, [lr, lc], zerosf, mask=m)
                return jnp.int32(0)

            lax.fori_loop(0, nch, ch_body, jnp.int32(0))
            return jnp.int32(0)

        lax.fori_loop(0, _NW, seg_apply, jnp.int32(0))

        pltpu.sync_copy(reg_v, out_h.at[pl.ds(row0, _BCELLS // _W)])
        return jnp.int32(0)

    lax.fori_loop(0, _KPW, bucket_body, jnp.int32(0))


@functools.cache
def _sc_sort():
    mesh = plsc.VectorSubcoreMesh(core_axis_name="c", subcore_axis_name="s")
    return pl.kernel(
        _sc_sort_body,
        out_type=(
            jax.ShapeDtypeStruct((_NW * _SSTR,), jnp.int32),
            jax.ShapeDtypeStruct((_NW * _BSTR,), jnp.int32),
        ),
        mesh=mesh,
        compiler_params=pltpu.CompilerParams(needs_layout_passes=False),
        scratch_types=[
            pltpu.VMEM((_GH,), jnp.int32),
            pltpu.VMEM((_GH,), jnp.int32),
            pltpu.VMEM((_GH,), jnp.int32),
            pltpu.VMEM((_GH,), jnp.int32),
            pltpu.VMEM((_GH,), jnp.float32),
            pltpu.VMEM((_GH,), jnp.float32),
            pltpu.VMEM((_NI,), jnp.int32),
            pltpu.VMEM((_NI,), jnp.int32),
            pltpu.VMEM((_NSUB,), jnp.int32),
            pltpu.VMEM((_BSTR,), jnp.int32),
            pltpu.SemaphoreType.DMA,
        ],
    )


@functools.cache
def _sc_apply():
    mesh = plsc.VectorSubcoreMesh(core_axis_name="c", subcore_axis_name="s")
    return pl.kernel(
        _sc_apply_body,
        out_type=jax.ShapeDtypeStruct((_H, _W), jnp.float32),
        mesh=mesh,
        compiler_params=pltpu.CompilerParams(needs_layout_passes=False),
        scratch_types=[
            pltpu.VMEM((_NW * 32,), jnp.int32),
            pltpu.VMEM((_BCELLS // _W, _W), jnp.float32),
            pltpu.VMEM((_NW, _CH), jnp.int32),
            pltpu.VMEM((_NW,), jnp.int32),
            pltpu.VMEM((_NW,), jnp.int32),
            pltpu.SemaphoreType.DMA,
            pltpu.SemaphoreType.DMA,
        ],
    )


def kernel(x, gens, pd):
    npad = _PPAD - _P
    zpad = jnp.zeros((npad,), jnp.int32)
    cols = [jnp.concatenate([gens[:, j], zpad]) for j in range(4)]
    pd_b = jnp.concatenate([pd[:, 0], jnp.zeros((npad,), jnp.float32)])
    pd_d = jnp.concatenate([pd[:, 1], jnp.ones((npad,), jnp.float32)])

    sorted_h, bounds_h = _sc_sort()(*cols, pd_b, pd_d)
    return _sc_apply()(x, sorted_h, bounds_h)


# speedup vs baseline: 4.4389x; 1.0991x over previous
import functools

import jax
import jax.numpy as jnp
from jax import lax
from jax.experimental import pallas as pl
from jax.experimental.pallas import tpu as pltpu
from jax.experimental.pallas import tpu_sc as plsc

_H = 4096
_W = 4096
_HW = _H * _W
_P = 500000
_THETA = 0.5

_NC = 2
_NS = 16
_NW = _NC * _NS

_PPAD = 503808
_G = _PPAD // _NW
_GH = _G // 2
_NI = 2 * _G

_NBUK = 512
_BSH = 15
_BCELLS = 1 << _BSH
_KPW = _NBUK // _NW
_NSUB = _NBUK * 16

_SSTR = _NI + 128
_BSTR = 528
_CH = 128


def _sc_sort_body(r0_h, c0_h, r1_h, c1_h, b_h, d_h, sorted_h, bounds_h,
                  r0_v, c0_v, r1_v, c1_v, b_v, d_v,
                  raw_v, srt_v, hist, bnd_v, sem):
    wid = lax.axis_index("s") * _NC + lax.axis_index("c")
    base = wid * _G

    iota = lax.iota(jnp.int32, 16)
    ones16 = jnp.ones((16,), jnp.int32)
    zero16 = jnp.zeros((16,), jnp.int32)

    def hz_body(i, _):
        for j in range(8):
            hist[pl.ds((i * 8 + j) * 16, 16)] = zero16
        return jnp.int32(0)

    lax.fori_loop(0, _NSUB // 128, hz_body, jnp.int32(0))

    stage = [(r0_h, r0_v), (c0_h, c0_v), (r1_h, r1_v),
             (c1_h, c1_v), (b_h, b_v), (d_h, d_v)]
    for ch in range(2):
        off = base + ch * _GH
        for src, dst in stage:
            pltpu.async_copy(src.at[pl.ds(off, _GH)], dst, sem)
        for src, dst in stage:
            pltpu.make_async_copy(src.at[pl.ds(off, _GH)], dst, sem).wait()

        def p1_body(i, _):
            for j in range(4):
                v = i * 4 + j
                sl = pl.ds(v * 16, 16)
                m = jnp.abs(d_v[sl] - b_v[sl]) <= _THETA
                f0 = jnp.where(m, r0_v[sl] * _W + c0_v[sl], -1)
                f1 = jnp.where(m, r1_v[sl] * _W + c1_v[sl], -1)
                raw_v[pl.ds(ch * 2 * _GH + v * 16, 16)] = f0
                raw_v[pl.ds(ch * 2 * _GH + _GH + v * 16, 16)] = f1
                s0 = lax.shift_right_arithmetic(f0, _BSH) * 16 + iota
                s1 = lax.shift_right_arithmetic(f1, _BSH) * 16 + iota
                plsc.addupdate_scatter(hist, [jnp.where(m, s0, 0)], ones16,
                                       mask=m)
                plsc.addupdate_scatter(hist, [jnp.where(m, s1, 0)], ones16,
                                       mask=m)
            return jnp.int32(0)

        lax.fori_loop(0, _GH // 64, p1_body, jnp.int32(0))

    def off_body(i, carry):
        for j in range(4):
            sl = pl.ds((i * 4 + j) * 16, 16)
            h = hist[sl]
            inc = plsc.cumsum(h)
            hist[sl] = inc - h + carry
            carry = carry + jnp.sum(h)
        return carry

    nvalid = lax.fori_loop(0, _NSUB // 64, off_body, jnp.int32(0))

    def bb_body(i, _):
        kvec = i * 16 + iota
        bnd_v[pl.ds(i * 16, 16)] = plsc.load_gather(hist, [kvec * 16])
        return jnp.int32(0)

    lax.fori_loop(0, _NBUK // 16, bb_body, jnp.int32(0))
    m0 = iota == 0
    plsc.store_scatter(bnd_v, [jnp.where(m0, _NBUK, 0)],
                       jnp.full((16,), 0, jnp.int32) + nvalid, mask=m0)

    def p2_body(i, _):
        for j in range(4):
            sl = pl.ds((i * 4 + j) * 16, 16)
            f = raw_v[sl]
            m = f >= 0
            sub = jnp.where(m, lax.shift_right_arithmetic(f, _BSH) * 16 + iota,
                            0)
            pos = plsc.load_gather(hist, [sub], mask=m)
            plsc.addupdate_scatter(hist, [sub], ones16, mask=m)
            plsc.store_scatter(srt_v, [jnp.where(m, pos, 0)], f, mask=m)
        return jnp.int32(0)

    lax.fori_loop(0, _NI // 64, p2_body, jnp.int32(0))

    pltpu.sync_copy(srt_v, sorted_h.at[pl.ds(wid * _SSTR, _NI)])
    pltpu.sync_copy(bnd_v, bounds_h.at[pl.ds(wid * _BSTR, _BSTR)])


def _sc_apply_body(x_h, sorted_h, bounds_h, out_h,
                   bnd_v, reg_a, reg_b, chk_v, st_v, en_v,
                   sem, semra, semrb, semoa, semob):
    wid = lax.axis_index("s") * _NC + lax.axis_index("c")
    kbase = wid * _KPW

    iota = lax.iota(jnp.int32, 16)
    zerosf = jnp.zeros((16,), jnp.float32)

    for s in range(_NW):
        pltpu.async_copy(bounds_h.at[pl.ds(s * _BSTR + kbase, 32)],
                         bnd_v.at[pl.ds(s * 32, 32)], sem)
    for s in range(_NW):
        pltpu.make_async_copy(bounds_h.at[pl.ds(s * _BSTR + kbase, 32)],
                              bnd_v.at[pl.ds(s * 32, 32)], sem).wait()

    def _extract(buf, vbase, p):
        v = buf[pl.ds(vbase + lax.bitwise_and(p, 16), 16)]
        return jnp.sum(jnp.where(iota == lax.bitwise_and(p, 15), v, 0))

    def _rows(k):
        return pl.multiple_of((kbase + k) * (_BCELLS // _W), 8)

    def _load(k, reg, semr):
        pltpu.async_copy(x_h.at[pl.ds(_rows(k), _BCELLS // _W)], reg, semr)

    def _wait_load(k, reg, semr):
        pltpu.make_async_copy(x_h.at[pl.ds(_rows(k), _BCELLS // _W)], reg,
                              semr).wait()

    def _store(k, reg, semo):
        pltpu.async_copy(reg, out_h.at[pl.ds(_rows(k), _BCELLS // _W)], semo)

    def _wait_store(k, reg, semo):
        pltpu.make_async_copy(reg, out_h.at[pl.ds(_rows(k), _BCELLS // _W)],
                              semo).wait()

    def _segfetch(k):
        def seg_issue(s, _):
            start = _extract(bnd_v, s * 32, k)
            end = _extract(bnd_v, s * 32, k + 1)
            astart = lax.bitwise_and(start, jnp.int32(-8))
            plsc.store_scatter(st_v, [jnp.where(iota == 0, s, 0)],
                               jnp.zeros((16,), jnp.int32) + start,
                               mask=iota == 0)
            plsc.store_scatter(en_v, [jnp.where(iota == 0, s, 0)],
                               jnp.zeros((16,), jnp.int32) + end,
                               mask=iota == 0)
            soff = pl.multiple_of(s * _SSTR + astart, 8)
            pltpu.async_copy(sorted_h.at[pl.ds(soff, _CH)],
                             chk_v.at[s], sem)
            return jnp.int32(0)

        lax.fori_loop(0, _NW, seg_issue, jnp.int32(0))

        def seg_wait(s, _):
            start = _extract(st_v, 0, s)
            astart = lax.bitwise_and(start, jnp.int32(-8))
            soff = pl.multiple_of(s * _SSTR + astart, 8)
            pltpu.make_async_copy(sorted_h.at[pl.ds(soff, _CH)],
                                  chk_v.at[s], sem).wait()
            return jnp.int32(0)

        lax.fori_loop(0, _NW, seg_wait, jnp.int32(0))

    def _apply(k, reg):
        rbase = (kbase + k) * _BCELLS

        def seg_apply(s, _):
            start = _extract(st_v, 0, s)
            end = _extract(en_v, 0, s)
            astart = lax.bitwise_and(start, jnp.int32(-8))
            nch = lax.shift_right_logical(end - astart + (_CH - 1), 7)

            def ch_body(c, _):
                coff = astart + c * _CH

                @pl.when(c > 0)
                def _():
                    soff = pl.multiple_of(s * _SSTR + coff, 8)
                    pltpu.async_copy(
                        sorted_h.at[pl.ds(soff, _CH)],
                        chk_v.at[s], sem).wait()

                for v in range(_CH // 16):
                    posa = coff + v * 16 + iota
                    f = chk_v[s, pl.ds(v * 16, 16)]
                    m = jnp.logical_and(posa >= start, posa < end)
                    loc = jnp.where(m, f - rbase, 0)
                    lr = lax.shift_right_logical(loc, 12)
                    lc = lax.bitwise_and(loc, _W - 1)
                    plsc.store_scatter(reg, [lr, lc], zerosf, mask=m)
                return jnp.int32(0)

            lax.fori_loop(0, nch, ch_body, jnp.int32(0))
            return jnp.int32(0)

        lax.fori_loop(0, _NW, seg_apply, jnp.int32(0))

    _load(0, reg_a, semra)

    def pair_body(p, _):
        k0 = 2 * p
        k1 = k0 + 1

        @pl.when(p > 0)
        def _():
            _wait_store(k1 - 2, reg_b, semob)

        _load(k1, reg_b, semrb)
        _segfetch(k0)
        _wait_load(k0, reg_a, semra)
        _apply(k0, reg_a)
        _store(k0, reg_a, semoa)

        _segfetch(k1)

        @pl.when(p < _KPW // 2 - 1)
        def _():
            _wait_store(k0, reg_a, semoa)
            _load(k0 + 2, reg_a, semra)

        _wait_load(k1, reg_b, semrb)
        _apply(k1, reg_b)
        _store(k1, reg_b, semob)
        return jnp.int32(0)

    lax.fori_loop(0, _KPW // 2, pair_body, jnp.int32(0))
    _wait_store(_KPW - 2, reg_a, semoa)
    _wait_store(_KPW - 1, reg_b, semob)


@functools.cache
def _sc_sort():
    mesh = plsc.VectorSubcoreMesh(core_axis_name="c", subcore_axis_name="s")
    return pl.kernel(
        _sc_sort_body,
        out_type=(
            jax.ShapeDtypeStruct((_NW * _SSTR,), jnp.int32),
            jax.ShapeDtypeStruct((_NW * _BSTR,), jnp.int32),
        ),
        mesh=mesh,
        compiler_params=pltpu.CompilerParams(needs_layout_passes=False),
        scratch_types=[
            pltpu.VMEM((_GH,), jnp.int32),
            pltpu.VMEM((_GH,), jnp.int32),
            pltpu.VMEM((_GH,), jnp.int32),
            pltpu.VMEM((_GH,), jnp.int32),
            pltpu.VMEM((_GH,), jnp.float32),
            pltpu.VMEM((_GH,), jnp.float32),
            pltpu.VMEM((_NI,), jnp.int32),
            pltpu.VMEM((_NI,), jnp.int32),
            pltpu.VMEM((_NSUB,), jnp.int32),
            pltpu.VMEM((_BSTR,), jnp.int32),
            pltpu.SemaphoreType.DMA,
        ],
    )


@functools.cache
def _sc_apply():
    mesh = plsc.VectorSubcoreMesh(core_axis_name="c", subcore_axis_name="s")
    return pl.kernel(
        _sc_apply_body,
        out_type=jax.ShapeDtypeStruct((_H, _W), jnp.float32),
        mesh=mesh,
        compiler_params=pltpu.CompilerParams(needs_layout_passes=False),
        scratch_types=[
            pltpu.VMEM((_NW * 32,), jnp.int32),
            pltpu.VMEM((_BCELLS // _W, _W), jnp.float32),
            pltpu.VMEM((_BCELLS // _W, _W), jnp.float32),
            pltpu.VMEM((_NW, _CH), jnp.int32),
            pltpu.VMEM((_NW,), jnp.int32),
            pltpu.VMEM((_NW,), jnp.int32),
            pltpu.SemaphoreType.DMA,
            pltpu.SemaphoreType.DMA,
            pltpu.SemaphoreType.DMA,
            pltpu.SemaphoreType.DMA,
            pltpu.SemaphoreType.DMA,
        ],
    )


def kernel(x, gens, pd):
    npad = _PPAD - _P
    zpad = jnp.zeros((npad,), jnp.int32)
    cols = [jnp.concatenate([gens[:, j], zpad]) for j in range(4)]
    pd_b = jnp.concatenate([pd[:, 0], jnp.zeros((npad,), jnp.float32)])
    pd_d = jnp.concatenate([pd[:, 1], jnp.ones((npad,), jnp.float32)])

    sorted_h, bounds_h = _sc_sort()(*cols, pd_b, pd_d)
    return _sc_apply()(x, sorted_h, bounds_h)
